# baseline (speedup 1.0000x reference)
import functools

import jax
import jax.numpy as jnp
from jax.experimental import pallas as pl
from jax.experimental.pallas import tpu as pltpu
from jax.experimental.pallas import tpu_sc as plsc

N_TOK = 2048
DIM = 2048
INTER = 1024
E = 8

NA = 2 * N_TOK
B = 256
M_MAX = 6144
NB = M_MAX // B
BT = 256
BI = 512
J = INTER // BI
SWIN = 16


def _fl(x):
    return x.astype(jnp.float32)


def _meta_body(l_ref, wd_ref, dest_ref, bexp_ref):
    logits = l_ref[...]
    m = jnp.max(logits, axis=1, keepdims=True)
    p = jnp.exp(logits - m)
    p = p / jnp.sum(p, axis=1, keepdims=True)
    iot = jax.lax.broadcasted_iota(jnp.int32, p.shape, 1)
    m1 = jnp.max(p, axis=1, keepdims=True)
    i1 = jnp.min(jnp.where(p == m1, iot, E), axis=1, keepdims=True)
    p2 = jnp.where(iot == i1, -jnp.inf, p)
    m2 = jnp.max(p2, axis=1, keepdims=True)
    i2 = jnp.min(jnp.where(p2 == m2, iot, E), axis=1, keepdims=True)
    wdense = jnp.where(iot == i1, m1, 0.0) + jnp.where(iot == i2, m2, 0.0)
    wd_ref[...] = wdense

    wT = jnp.transpose(wdense)
    si = jax.lax.broadcasted_iota(jnp.int32, (E, N_TOK), 0)
    t1 = jnp.max(wT, axis=0, keepdims=True)
    j1 = jnp.min(jnp.where(wT == t1, si, E), axis=0, keepdims=True)
    wr = jnp.where(si == j1, -1.0, wT)
    t2 = jnp.max(wr, axis=0, keepdims=True)
    j2 = jnp.min(jnp.where(wr == t2, si, E), axis=0, keepdims=True)
    oh1 = _fl(si == j1)
    oh2 = _fl(si == j2)
    A = jnp.concatenate([oh1, oh2], axis=1)

    A3 = A.reshape(E, NA // 128, 128)
    r128 = jax.lax.broadcasted_iota(jnp.int32, (128, 128), 0)
    c128 = jax.lax.broadcasted_iota(jnp.int32, (128, 128), 1)
    tri128 = _fl(r128 < c128)
    within = jax.lax.dot_general(A3, tri128, (((2,), (0,)), ((), ())),
                                 preferred_element_type=jnp.float32)
    cs = jnp.sum(A3, axis=2)
    nch = NA // 128
    rch = jax.lax.broadcasted_iota(jnp.int32, (nch, nch), 0)
    cch = jax.lax.broadcasted_iota(jnp.int32, (nch, nch), 1)
    trich = _fl(rch < cch)
    cpref = jax.lax.dot_general(cs, trich, (((1,), (0,)), ((), ())),
                                preferred_element_type=jnp.float32)
    rank = (within + cpref[:, :, None]).reshape(E, NA)

    counts = jnp.sum(A, axis=1, keepdims=True)
    pc = jnp.floor((counts + (B - 1)) / B) * B
    re8 = jax.lax.broadcasted_iota(jnp.int32, (E, E), 0)
    ce8 = jax.lax.broadcasted_iota(jnp.int32, (E, E), 1)
    lt8 = _fl(ce8 < re8)
    offs = jax.lax.dot_general(lt8, pc, (((1,), (0,)), ((), ())),
                               preferred_element_type=jnp.float32)
    dest = rank + offs
    desta = jnp.sum(A * dest, axis=0, keepdims=True)
    dest_ref[...] = desta.astype(jnp.int32)

    li = jax.lax.broadcasted_iota(jnp.int32, (1, 32), 1)
    bstart = _fl(li) * B
    nbelow = jnp.sum(_fl(offs <= bstart), axis=0, keepdims=True)
    bexp = nbelow - 1.0
    nact = jnp.sum(pc) / B
    row = jnp.where(li == NB, nact, bexp)
    bexp_ref[...] = row.astype(jnp.int32)


def _meta(logits):
    return pl.pallas_call(
        _meta_body,
        grid=(1,),
        in_specs=[pl.BlockSpec((N_TOK, E), lambda i: (0, 0))],
        out_specs=[
            pl.BlockSpec((N_TOK, E), lambda i: (0, 0)),
            pl.BlockSpec((1, NA), lambda i: (0, 0)),
            pl.BlockSpec((1, 32), lambda i: (0, 0)),
        ],
        out_shape=[
            jax.ShapeDtypeStruct((N_TOK, E), jnp.float32),
            jax.ShapeDtypeStruct((1, NA), jnp.int32),
            jax.ShapeDtypeStruct((1, 32), jnp.int32),
        ],
    )(logits)


IWIN = 128
NCH = IWIN // SWIN
DIM32 = DIM // 2


def _vmesh():
    return plsc.VectorSubcoreMesh(
        core_axis_name="core", subcore_axis_name="subcore")


def _sc_move(data, src_idx, dst_idx, out_rows):
    @functools.partial(
        pl.kernel,
        out_type=jax.ShapeDtypeStruct((out_rows, DIM32), jnp.int32),
        mesh=_vmesh(),
        scratch_types=[pltpu.VMEM((SWIN, DIM32), jnp.int32)],
    )
    def k(x_hbm, src_hbm, dst_hbm, o_hbm, buf):
        def body(src_vmem, dst_vmem):
            @pl.loop(0, NCH)
            def _(c):
                sl = pl.ds(c * SWIN, SWIN)
                pltpu.sync_copy(x_hbm.at[src_vmem.at[0, sl]], buf)
                pltpu.sync_copy(buf, o_hbm.at[dst_vmem.at[0, sl]])

        pltpu.emit_pipeline(
            body,
            grid=(NA // IWIN,),
            in_specs=[
                pl.BlockSpec((1, IWIN), lambda i: (0, i)),
                pl.BlockSpec((1, IWIN), lambda i: (0, i)),
            ],
            out_specs=[],
            core_axis_name=("core", "subcore"),
            dimension_semantics=(pltpu.PARALLEL,),
        )(src_hbm, dst_hbm)

    return k(data, src_idx, dst_idx)


def _as_i32(rows_bf16):
    m, d = rows_bf16.shape
    return jax.lax.bitcast_convert_type(
        rows_bf16.reshape(m, d // 2, 2), jnp.int32)


def _as_bf16(rows_i32):
    m, d32 = rows_i32.shape
    return jax.lax.bitcast_convert_type(
        rows_i32, jnp.bfloat16).reshape(m, 2 * d32)


def _sc_dispatch(xb, toka, desta):
    return _as_bf16(_sc_move(_as_i32(xb), toka, desta, M_MAX))


def _sc_gather(ys, desta):
    iota = jnp.arange(NA, dtype=jnp.int32).reshape(1, NA)
    return _as_bf16(_sc_move(_as_i32(ys), desta, iota, NA))


def _grouped_body(s_ref, xs_ref, w1_ref, w3_ref, w2_ref, ys_ref,
                  w1b, w3b, w2b):
    jj = pl.program_id(0)
    b = pl.program_id(1)
    eb = s_ref[b]
    prev = s_ref[jnp.maximum(b - 1, 0)]
    changed = jnp.logical_or(b == 0, eb != prev)
    active = b < s_ref[NB]

    @pl.when(jnp.logical_and(changed, active))
    def _():
        w1b[...] = w1_ref[0].astype(jnp.bfloat16)
        w3b[...] = w3_ref[0].astype(jnp.bfloat16)
        w2b[...] = w2_ref[0].astype(jnp.bfloat16)

    @pl.when(active)
    def _():
        x = xs_ref[...]
        h1 = jax.lax.dot_general(x, w1b[...], (((1,), (1,)), ((), ())),
                                 preferred_element_type=jnp.float32)
        h3 = jax.lax.dot_general(x, w3b[...], (((1,), (1,)), ((), ())),
                                 preferred_element_type=jnp.float32)
        g = (jax.nn.silu(h1) * h3).astype(jnp.bfloat16)
        o = jax.lax.dot_general(g, w2b[...], (((1,), (1,)), ((), ())),
                                preferred_element_type=jnp.float32)
        rows = pl.ds(b * B, B)

        @pl.when(jj == 0)
        def _():
            ys_ref[rows, :] = o.astype(jnp.bfloat16)

        @pl.when(jj == 1)
        def _():
            ys_ref[rows, :] = (ys_ref[rows, :].astype(jnp.float32) + o
                               ).astype(jnp.bfloat16)


def _grouped(scal, xs, W1, W3, W2):
    grid_spec = pltpu.PrefetchScalarGridSpec(
        num_scalar_prefetch=1,
        grid=(J, NB),
        in_specs=[
            pl.BlockSpec((B, DIM), lambda j, b, s: (b, 0)),
            pl.BlockSpec((1, BI, DIM), lambda j, b, s: (s[b], j, 0)),
            pl.BlockSpec((1, BI, DIM), lambda j, b, s: (s[b], j, 0)),
            pl.BlockSpec((1, DIM, BI), lambda j, b, s: (s[b], 0, j)),
        ],
        out_specs=pl.BlockSpec((M_MAX, DIM), lambda j, b, s: (0, 0)),
        scratch_shapes=[
            pltpu.VMEM((BI, DIM), jnp.bfloat16),
            pltpu.VMEM((BI, DIM), jnp.bfloat16),
            pltpu.VMEM((DIM, BI), jnp.bfloat16),
        ],
    )
    return pl.pallas_call(
        _grouped_body,
        grid_spec=grid_spec,
        out_shape=jax.ShapeDtypeStruct((M_MAX, DIM), jnp.bfloat16),
    )(scal, xs, W1, W3, W2)


def _shared_body(xb_ref, w1_ref, w3_ref, w2_ref, z_ref, w1b, w3b, w2b):
    t = pl.program_id(0)

    @pl.when(t == 0)
    def _():
        w1b[...] = w1_ref[...].astype(jnp.bfloat16)
        w3b[...] = w3_ref[...].astype(jnp.bfloat16)
        w2b[...] = w2_ref[...].astype(jnp.bfloat16)

    xt = xb_ref[pl.ds(t * BT, BT), :]
    h1 = jax.lax.dot_general(xt, w1b[...], (((1,), (1,)), ((), ())),
                             preferred_element_type=jnp.float32)
    h3 = jax.lax.dot_general(xt, w3b[...], (((1,), (1,)), ((), ())),
                             preferred_element_type=jnp.float32)
    g = (jax.nn.silu(h1) * h3).astype(jnp.bfloat16)
    o = jax.lax.dot_general(g, w2b[...], (((1,), (1,)), ((), ())),
                            preferred_element_type=jnp.float32)
    z_ref[...] = o.astype(jnp.bfloat16)


def _shared(xb, Ws1, Ws3, Ws2, jj):
    return pl.pallas_call(
        _shared_body,
        grid=(N_TOK // BT,),
        in_specs=[
            pl.BlockSpec((N_TOK, DIM), lambda t: (0, 0)),
            pl.BlockSpec((BI, DIM), lambda t, jj=jj: (jj, 0)),
            pl.BlockSpec((BI, DIM), lambda t, jj=jj: (jj, 0)),
            pl.BlockSpec((DIM, BI), lambda t, jj=jj: (0, jj)),
        ],
        out_specs=pl.BlockSpec((BT, DIM), lambda t: (t, 0)),
        out_shape=jax.ShapeDtypeStruct((N_TOK, DIM), jnp.bfloat16),
        scratch_shapes=[
            pltpu.VMEM((BI, DIM), jnp.bfloat16),
            pltpu.VMEM((BI, DIM), jnp.bfloat16),
            pltpu.VMEM((DIM, BI), jnp.bfloat16),
        ],
    )(xb, Ws1, Ws3, Ws2)


def _combine_body(wd_ref, z0_ref, z1_ref, y1_ref, y2_ref, out_ref):
    wd = wd_ref[...]
    w1 = jnp.max(wd, axis=1, keepdims=True)
    w2 = jnp.sum(wd, axis=1, keepdims=True) - w1
    z = z0_ref[...].astype(jnp.float32) + z1_ref[...].astype(jnp.float32)
    out_ref[...] = (z + w1 * y1_ref[...].astype(jnp.float32)
                    + w2 * y2_ref[...].astype(jnp.float32))


def _combine(wd, z0, z1, yg):
    nt = N_TOK // BT
    return pl.pallas_call(
        _combine_body,
        grid=(nt,),
        in_specs=[
            pl.BlockSpec((BT, E), lambda t: (t, 0)),
            pl.BlockSpec((BT, DIM), lambda t: (t, 0)),
            pl.BlockSpec((BT, DIM), lambda t: (t, 0)),
            pl.BlockSpec((BT, DIM), lambda t: (t, 0)),
            pl.BlockSpec((BT, DIM), lambda t, nt=nt: (t + nt, 0)),
        ],
        out_specs=pl.BlockSpec((BT, DIM), lambda t: (t, 0)),
        out_shape=jax.ShapeDtypeStruct((N_TOK, DIM), jnp.float32),
    )(wd, z0, z1, yg, yg)


def kernel(x, gate_w, W1, W2, W3, Ws1, Ws2, Ws3):
    xb = x.astype(jnp.bfloat16)
    logits = x @ gate_w.T
    wd, desta, bexp = _meta(logits)
    scal = bexp.reshape(32)
    toka = jnp.concatenate(
        [jnp.arange(N_TOK, dtype=jnp.int32)] * 2).reshape(1, NA)
    xs = _sc_dispatch(xb, toka, desta)
    z0 = _shared(xb, Ws1, Ws3, Ws2, 0)
    ys = _grouped(scal, xs, W1, W3, W2)
    z1 = _shared(xb, Ws1, Ws3, Ws2, 1)
    yg = _sc_gather(ys, desta)
    return _combine(wd, z0, z1, yg)


# speedup vs baseline: 2.0373x; 2.0373x over previous
import functools

import jax
import jax.numpy as jnp
from jax.experimental import pallas as pl
from jax.experimental.pallas import tpu as pltpu
from jax.experimental.pallas import tpu_sc as plsc

N_TOK = 2048
DIM = 2048
INTER = 1024
E = 8

NA = 2 * N_TOK
B = 256
M_MAX = 6144
NB = M_MAX // B
BT = 256
BI = 512
J = INTER // BI
SWIN = 16


def _fl(x):
    return x.astype(jnp.float32)


def _meta_body(l_ref, wd_ref, dest_ref, bexp_ref):
    logits = l_ref[...]
    m = jnp.max(logits, axis=1, keepdims=True)
    p = jnp.exp(logits - m)
    p = p / jnp.sum(p, axis=1, keepdims=True)
    iot = jax.lax.broadcasted_iota(jnp.int32, p.shape, 1)
    m1 = jnp.max(p, axis=1, keepdims=True)
    i1 = jnp.min(jnp.where(p == m1, iot, E), axis=1, keepdims=True)
    p2 = jnp.where(iot == i1, -jnp.inf, p)
    m2 = jnp.max(p2, axis=1, keepdims=True)
    i2 = jnp.min(jnp.where(p2 == m2, iot, E), axis=1, keepdims=True)
    wdense = jnp.where(iot == i1, m1, 0.0) + jnp.where(iot == i2, m2, 0.0)
    wd_ref[...] = wdense

    wT = jnp.transpose(wdense)
    si = jax.lax.broadcasted_iota(jnp.int32, (E, N_TOK), 0)
    t1 = jnp.max(wT, axis=0, keepdims=True)
    j1 = jnp.min(jnp.where(wT == t1, si, E), axis=0, keepdims=True)
    wr = jnp.where(si == j1, -1.0, wT)
    t2 = jnp.max(wr, axis=0, keepdims=True)
    j2 = jnp.min(jnp.where(wr == t2, si, E), axis=0, keepdims=True)
    oh1 = _fl(si == j1)
    oh2 = _fl(si == j2)
    A = jnp.concatenate([oh1, oh2], axis=1)

    A3 = A.reshape(E, NA // 128, 128)
    r128 = jax.lax.broadcasted_iota(jnp.int32, (128, 128), 0)
    c128 = jax.lax.broadcasted_iota(jnp.int32, (128, 128), 1)
    tri128 = _fl(r128 < c128)
    within = jax.lax.dot_general(A3, tri128, (((2,), (0,)), ((), ())),
                                 preferred_element_type=jnp.float32)
    cs = jnp.sum(A3, axis=2)
    nch = NA // 128
    rch = jax.lax.broadcasted_iota(jnp.int32, (nch, nch), 0)
    cch = jax.lax.broadcasted_iota(jnp.int32, (nch, nch), 1)
    trich = _fl(rch < cch)
    cpref = jax.lax.dot_general(cs, trich, (((1,), (0,)), ((), ())),
                                preferred_element_type=jnp.float32)
    rank = (within + cpref[:, :, None]).reshape(E, NA)

    counts = jnp.sum(A, axis=1, keepdims=True)
    pc = jnp.floor((counts + (B - 1)) / B) * B
    re8 = jax.lax.broadcasted_iota(jnp.int32, (E, E), 0)
    ce8 = jax.lax.broadcasted_iota(jnp.int32, (E, E), 1)
    lt8 = _fl(ce8 < re8)
    offs = jax.lax.dot_general(lt8, pc, (((1,), (0,)), ((), ())),
                               preferred_element_type=jnp.float32)
    dest = rank + offs
    desta = jnp.sum(A * dest, axis=0, keepdims=True)
    dest_ref[...] = desta.astype(jnp.int32)

    li = jax.lax.broadcasted_iota(jnp.int32, (1, 32), 1)
    bstart = _fl(li) * B
    nbelow = jnp.sum(_fl(offs <= bstart), axis=0, keepdims=True)
    bexp = nbelow - 1.0
    nact = jnp.sum(pc) / B
    row = jnp.where(li == NB, nact, bexp)
    bexp_ref[...] = row.astype(jnp.int32)


def _meta(logits):
    return pl.pallas_call(
        _meta_body,
        grid=(1,),
        in_specs=[pl.BlockSpec((N_TOK, E), lambda i: (0, 0))],
        out_specs=[
            pl.BlockSpec((N_TOK, E), lambda i: (0, 0)),
            pl.BlockSpec((1, NA), lambda i: (0, 0)),
            pl.BlockSpec((1, 32), lambda i: (0, 0)),
        ],
        out_shape=[
            jax.ShapeDtypeStruct((N_TOK, E), jnp.float32),
            jax.ShapeDtypeStruct((1, NA), jnp.int32),
            jax.ShapeDtypeStruct((1, 32), jnp.int32),
        ],
    )(logits)


IWIN = 128
NCH = IWIN // SWIN


def _vmesh():
    return plsc.VectorSubcoreMesh(
        core_axis_name="core", subcore_axis_name="subcore")


def _sc_move(data, src_idx, dst_idx, out_rows):
    @functools.partial(
        pl.kernel,
        out_type=jax.ShapeDtypeStruct((out_rows, DIM), jnp.int32),
        mesh=_vmesh(),
        scratch_types=[pltpu.VMEM((SWIN, DIM), jnp.int32)],
    )
    def k(x_hbm, src_hbm, dst_hbm, o_hbm, buf):
        def body(src_vmem, dst_vmem):
            @pl.loop(0, NCH)
            def _(c):
                sl = pl.ds(c * SWIN, SWIN)
                pltpu.sync_copy(x_hbm.at[src_vmem.at[0, sl]], buf)
                pltpu.sync_copy(buf, o_hbm.at[dst_vmem.at[0, sl]])

        pltpu.emit_pipeline(
            body,
            grid=(NA // IWIN,),
            in_specs=[
                pl.BlockSpec((1, IWIN), lambda i: (0, i)),
                pl.BlockSpec((1, IWIN), lambda i: (0, i)),
            ],
            out_specs=[],
            core_axis_name=("core", "subcore"),
            dimension_semantics=(pltpu.PARALLEL,),
        )(src_hbm, dst_hbm)

    return k(data, src_idx, dst_idx)


def _as_i32(rows_f32):
    return jax.lax.bitcast_convert_type(rows_f32, jnp.int32)


def _as_f32(rows_i32):
    return jax.lax.bitcast_convert_type(rows_i32, jnp.float32)


def _sc_dispatch(x, toka, desta):
    return _as_f32(_sc_move(_as_i32(x), toka, desta, M_MAX))


def _sc_gather(ys, desta):
    iota = jnp.arange(NA, dtype=jnp.int32).reshape(1, NA)
    return _as_f32(_sc_move(_as_i32(ys), desta, iota, NA))


def _grouped_body(s_ref, xs_ref, w1_ref, w3_ref, w2_ref, ys0_ref, ys1_ref,
                  w1b, w3b, w2b):
    jj = pl.program_id(0)
    b = pl.program_id(1)
    eb = s_ref[b]
    prev = s_ref[jnp.maximum(b - 1, 0)]
    changed = jnp.logical_or(b == 0, eb != prev)
    active = b < s_ref[NB]

    @pl.when(jnp.logical_and(changed, active))
    def _():
        w1b[...] = w1_ref[0].astype(jnp.bfloat16)
        w3b[...] = w3_ref[0].astype(jnp.bfloat16)
        w2b[...] = w2_ref[0].astype(jnp.bfloat16)

    @pl.when(active)
    def _():
        x = xs_ref[...].astype(jnp.bfloat16)
        h1 = jax.lax.dot_general(x, w1b[...], (((1,), (1,)), ((), ())),
                                 preferred_element_type=jnp.float32)
        h3 = jax.lax.dot_general(x, w3b[...], (((1,), (1,)), ((), ())),
                                 preferred_element_type=jnp.float32)
        g = (jax.nn.silu(h1) * h3).astype(jnp.bfloat16)
        o = jax.lax.dot_general(g, w2b[...], (((1,), (1,)), ((), ())),
                                preferred_element_type=jnp.float32)

        @pl.when(jj == 0)
        def _():
            ys0_ref[...] = o

        @pl.when(jj == 1)
        def _():
            ys1_ref[...] = o


def _grouped(scal, xs, W1, W3, W2):
    grid_spec = pltpu.PrefetchScalarGridSpec(
        num_scalar_prefetch=1,
        grid=(J, NB),
        in_specs=[
            pl.BlockSpec((B, DIM), lambda j, b, s: (b, 0)),
            pl.BlockSpec((1, BI, DIM), lambda j, b, s: (s[b], j, 0)),
            pl.BlockSpec((1, BI, DIM), lambda j, b, s: (s[b], j, 0)),
            pl.BlockSpec((1, DIM, BI), lambda j, b, s: (s[b], 0, j)),
        ],
        out_specs=[
            pl.BlockSpec((B, DIM),
                         lambda j, b, s: (jnp.where(j == 0, b, NB), 0)),
            pl.BlockSpec((B, DIM),
                         lambda j, b, s: (jnp.where(j == 1, b, NB), 0)),
        ],
        scratch_shapes=[
            pltpu.VMEM((BI, DIM), jnp.bfloat16),
            pltpu.VMEM((BI, DIM), jnp.bfloat16),
            pltpu.VMEM((DIM, BI), jnp.bfloat16),
        ],
    )
    return pl.pallas_call(
        _grouped_body,
        grid_spec=grid_spec,
        out_shape=[
            jax.ShapeDtypeStruct((M_MAX + B, DIM), jnp.float32),
            jax.ShapeDtypeStruct((M_MAX + B, DIM), jnp.float32),
        ],
    )(scal, xs, W1, W3, W2)


def _shared_body(xb_ref, w1_ref, w3_ref, w2_ref, z_ref, w1b, w3b, w2b):
    t = pl.program_id(0)

    @pl.when(t == 0)
    def _():
        w1b[...] = w1_ref[...].astype(jnp.bfloat16)
        w3b[...] = w3_ref[...].astype(jnp.bfloat16)
        w2b[...] = w2_ref[...].astype(jnp.bfloat16)

    xt = xb_ref[pl.ds(t * BT, BT), :]
    h1 = jax.lax.dot_general(xt, w1b[...], (((1,), (1,)), ((), ())),
                             preferred_element_type=jnp.float32)
    h3 = jax.lax.dot_general(xt, w3b[...], (((1,), (1,)), ((), ())),
                             preferred_element_type=jnp.float32)
    g = (jax.nn.silu(h1) * h3).astype(jnp.bfloat16)
    o = jax.lax.dot_general(g, w2b[...], (((1,), (1,)), ((), ())),
                            preferred_element_type=jnp.float32)
    z_ref[...] = o.astype(jnp.bfloat16)


def _shared(xb, Ws1, Ws3, Ws2, jj):
    return pl.pallas_call(
        _shared_body,
        grid=(N_TOK // BT,),
        in_specs=[
            pl.BlockSpec((N_TOK, DIM), lambda t: (0, 0)),
            pl.BlockSpec((BI, DIM), lambda t, jj=jj: (jj, 0)),
            pl.BlockSpec((BI, DIM), lambda t, jj=jj: (jj, 0)),
            pl.BlockSpec((DIM, BI), lambda t, jj=jj: (0, jj)),
        ],
        out_specs=pl.BlockSpec((BT, DIM), lambda t: (t, 0)),
        out_shape=jax.ShapeDtypeStruct((N_TOK, DIM), jnp.bfloat16),
        scratch_shapes=[
            pltpu.VMEM((BI, DIM), jnp.bfloat16),
            pltpu.VMEM((BI, DIM), jnp.bfloat16),
            pltpu.VMEM((DIM, BI), jnp.bfloat16),
        ],
    )(xb, Ws1, Ws3, Ws2)


def _combine_body(wd_ref, z0_ref, z1_ref, g0a_ref, g0b_ref, g1a_ref,
                  g1b_ref, out_ref):
    wd = wd_ref[...]
    w1 = jnp.max(wd, axis=1, keepdims=True)
    w2 = jnp.sum(wd, axis=1, keepdims=True) - w1
    z = z0_ref[...].astype(jnp.float32) + z1_ref[...].astype(jnp.float32)
    y1 = g0a_ref[...] + g1a_ref[...]
    y2 = g0b_ref[...] + g1b_ref[...]
    out_ref[...] = z + w1 * y1 + w2 * y2


def _combine(wd, z0, z1, g0, g1):
    nt = N_TOK // BT
    return pl.pallas_call(
        _combine_body,
        grid=(nt,),
        in_specs=[
            pl.BlockSpec((BT, E), lambda t: (t, 0)),
            pl.BlockSpec((BT, DIM), lambda t: (t, 0)),
            pl.BlockSpec((BT, DIM), lambda t: (t, 0)),
            pl.BlockSpec((BT, DIM), lambda t: (t, 0)),
            pl.BlockSpec((BT, DIM), lambda t, nt=nt: (t + nt, 0)),
            pl.BlockSpec((BT, DIM), lambda t: (t, 0)),
            pl.BlockSpec((BT, DIM), lambda t, nt=nt: (t + nt, 0)),
        ],
        out_specs=pl.BlockSpec((BT, DIM), lambda t: (t, 0)),
        out_shape=jax.ShapeDtypeStruct((N_TOK, DIM), jnp.float32),
    )(wd, z0, z1, g0, g0, g1, g1)


def kernel(x, gate_w, W1, W2, W3, Ws1, Ws2, Ws3):
    xb = x.astype(jnp.bfloat16)
    logits = x @ gate_w.T
    wd, desta, bexp = _meta(logits)
    scal = bexp.reshape(32)
    toka = jnp.concatenate(
        [jnp.arange(N_TOK, dtype=jnp.int32)] * 2).reshape(1, NA)
    xs = _sc_dispatch(x, toka, desta)
    z0 = _shared(xb, Ws1, Ws3, Ws2, 0)
    ys0, ys1 = _grouped(scal, xs, W1, W3, W2)
    z1 = _shared(xb, Ws1, Ws3, Ws2, 1)
    g0 = _sc_gather(ys0, desta)
    g1 = _sc_gather(ys1, desta)
    return _combine(wd, z0, z1, g0, g1)


# speedup vs baseline: 2.9719x; 1.4587x over previous
import functools

import jax
import jax.numpy as jnp
from jax.experimental import pallas as pl
from jax.experimental.pallas import tpu as pltpu
from jax.experimental.pallas import tpu_sc as plsc

N_TOK = 2048
DIM = 2048
INTER = 1024
E = 8

NA = 2 * N_TOK
B = 256
M_MAX = 6144
NB = M_MAX // B
BT = 256
BI = 512
J = INTER // BI
SWIN = 16


def _fl(x):
    return x.astype(jnp.float32)


def _meta_body(l_ref, wd_ref, dest_ref, bexp_ref):
    logits = l_ref[...]
    m = jnp.max(logits, axis=1, keepdims=True)
    p = jnp.exp(logits - m)
    p = p / jnp.sum(p, axis=1, keepdims=True)
    iot = jax.lax.broadcasted_iota(jnp.int32, p.shape, 1)
    m1 = jnp.max(p, axis=1, keepdims=True)
    i1 = jnp.min(jnp.where(p == m1, iot, E), axis=1, keepdims=True)
    p2 = jnp.where(iot == i1, -jnp.inf, p)
    m2 = jnp.max(p2, axis=1, keepdims=True)
    i2 = jnp.min(jnp.where(p2 == m2, iot, E), axis=1, keepdims=True)
    wdense = jnp.where(iot == i1, m1, 0.0) + jnp.where(iot == i2, m2, 0.0)
    wd_ref[...] = wdense

    wT = jnp.transpose(wdense)
    si = jax.lax.broadcasted_iota(jnp.int32, (E, N_TOK), 0)
    t1 = jnp.max(wT, axis=0, keepdims=True)
    j1 = jnp.min(jnp.where(wT == t1, si, E), axis=0, keepdims=True)
    wr = jnp.where(si == j1, -1.0, wT)
    t2 = jnp.max(wr, axis=0, keepdims=True)
    j2 = jnp.min(jnp.where(wr == t2, si, E), axis=0, keepdims=True)
    oh1 = _fl(si == j1)
    oh2 = _fl(si == j2)
    A = jnp.concatenate([oh1, oh2], axis=1)

    A3 = A.reshape(E, NA // 128, 128)
    r128 = jax.lax.broadcasted_iota(jnp.int32, (128, 128), 0)
    c128 = jax.lax.broadcasted_iota(jnp.int32, (128, 128), 1)
    tri128 = _fl(r128 < c128)
    within = jax.lax.dot_general(A3, tri128, (((2,), (0,)), ((), ())),
                                 preferred_element_type=jnp.float32)
    cs = jnp.sum(A3, axis=2)
    nch = NA // 128
    rch = jax.lax.broadcasted_iota(jnp.int32, (nch, nch), 0)
    cch = jax.lax.broadcasted_iota(jnp.int32, (nch, nch), 1)
    trich = _fl(rch < cch)
    cpref = jax.lax.dot_general(cs, trich, (((1,), (0,)), ((), ())),
                                preferred_element_type=jnp.float32)
    rank = (within + cpref[:, :, None]).reshape(E, NA)

    counts = jnp.sum(A, axis=1, keepdims=True)
    pc = jnp.floor((counts + (B - 1)) / B) * B
    re8 = jax.lax.broadcasted_iota(jnp.int32, (E, E), 0)
    ce8 = jax.lax.broadcasted_iota(jnp.int32, (E, E), 1)
    lt8 = _fl(ce8 < re8)
    offs = jax.lax.dot_general(lt8, pc, (((1,), (0,)), ((), ())),
                               preferred_element_type=jnp.float32)
    dest = rank + offs
    desta = jnp.sum(A * dest, axis=0, keepdims=True)
    dest_ref[...] = desta.astype(jnp.int32)

    li = jax.lax.broadcasted_iota(jnp.int32, (1, 32), 1)
    bstart = _fl(li) * B
    nbelow = jnp.sum(_fl(offs <= bstart), axis=0, keepdims=True)
    bexp = nbelow - 1.0
    nact = jnp.sum(pc) / B
    row = jnp.where(li == NB, nact, bexp)
    bexp_ref[...] = row.astype(jnp.int32)


def _meta(logits):
    return pl.pallas_call(
        _meta_body,
        grid=(1,),
        in_specs=[pl.BlockSpec((N_TOK, E), lambda i: (0, 0))],
        out_specs=[
            pl.BlockSpec((N_TOK, E), lambda i: (0, 0)),
            pl.BlockSpec((1, NA), lambda i: (0, 0)),
            pl.BlockSpec((1, 32), lambda i: (0, 0)),
        ],
        out_shape=[
            jax.ShapeDtypeStruct((N_TOK, E), jnp.float32),
            jax.ShapeDtypeStruct((1, NA), jnp.int32),
            jax.ShapeDtypeStruct((1, 32), jnp.int32),
        ],
    )(logits)


IWIN = 128
NCH = IWIN // SWIN


def _vmesh():
    return plsc.VectorSubcoreMesh(
        core_axis_name="core", subcore_axis_name="subcore")


def _sc_move(data, src_idx, dst_idx, out_rows):
    @functools.partial(
        pl.kernel,
        out_type=jax.ShapeDtypeStruct((out_rows, DIM), jnp.float32),
        mesh=_vmesh(),
        scratch_types=[pltpu.VMEM((SWIN, DIM), jnp.float32)],
    )
    def k(x_hbm, src_hbm, dst_hbm, o_hbm, buf):
        def body(src_vmem, dst_vmem):
            @pl.loop(0, NCH)
            def _(c):
                sl = pl.ds(c * SWIN, SWIN)
                pltpu.sync_copy(x_hbm.at[src_vmem.at[0, sl]], buf)
                pltpu.sync_copy(buf, o_hbm.at[dst_vmem.at[0, sl]])

        pltpu.emit_pipeline(
            body,
            grid=(NA // IWIN,),
            in_specs=[
                pl.BlockSpec((1, IWIN), lambda i: (0, i)),
                pl.BlockSpec((1, IWIN), lambda i: (0, i)),
            ],
            out_specs=[],
            core_axis_name=("core", "subcore"),
            dimension_semantics=(pltpu.PARALLEL,),
        )(src_hbm, dst_hbm)

    return k(data, src_idx, dst_idx)


def _sc_dispatch(x, toka, desta):
    return _sc_move(x, toka, desta, M_MAX)


def _sc_gather(ys, desta):
    iota = jnp.arange(NA, dtype=jnp.int32).reshape(1, NA)
    return _sc_move(ys, desta, iota, NA)


def _grouped_body(s_ref, xs_ref, w1_ref, w3_ref, w2_ref, ys0_ref, ys1_ref,
                  w1b, w3b, w2b):
    jj = pl.program_id(0)
    b = pl.program_id(1)
    eb = s_ref[b]
    prev = s_ref[jnp.maximum(b - 1, 0)]
    changed = jnp.logical_or(b == 0, eb != prev)
    active = b < s_ref[NB]

    @pl.when(jnp.logical_and(changed, active))
    def _():
        w1b[...] = w1_ref[0].astype(jnp.bfloat16)
        w3b[...] = w3_ref[0].astype(jnp.bfloat16)
        w2b[...] = w2_ref[0].astype(jnp.bfloat16)

    @pl.when(active)
    def _():
        x = xs_ref[...].astype(jnp.bfloat16)
        h1 = jax.lax.dot_general(x, w1b[...], (((1,), (1,)), ((), ())),
                                 preferred_element_type=jnp.float32)
        h3 = jax.lax.dot_general(x, w3b[...], (((1,), (1,)), ((), ())),
                                 preferred_element_type=jnp.float32)
        g = (jax.nn.silu(h1) * h3).astype(jnp.bfloat16)
        o = jax.lax.dot_general(g, w2b[...], (((1,), (1,)), ((), ())),
                                preferred_element_type=jnp.float32)

        @pl.when(jj == 0)
        def _():
            ys0_ref[...] = o

        @pl.when(jj == 1)
        def _():
            ys1_ref[...] = o


def _grouped(scal, xs, W1, W3, W2):
    grid_spec = pltpu.PrefetchScalarGridSpec(
        num_scalar_prefetch=1,
        grid=(J, NB),
        in_specs=[
            pl.BlockSpec((B, DIM), lambda j, b, s: (b, 0)),
            pl.BlockSpec((1, BI, DIM), lambda j, b, s: (s[b], j, 0)),
            pl.BlockSpec((1, BI, DIM), lambda j, b, s: (s[b], j, 0)),
            pl.BlockSpec((1, DIM, BI), lambda j, b, s: (s[b], 0, j)),
        ],
        out_specs=[
            pl.BlockSpec((B, DIM),
                         lambda j, b, s: (jnp.where(j == 0, b, NB), 0)),
            pl.BlockSpec((B, DIM),
                         lambda j, b, s: (jnp.where(j == 1, b, NB), 0)),
        ],
        scratch_shapes=[
            pltpu.VMEM((BI, DIM), jnp.bfloat16),
            pltpu.VMEM((BI, DIM), jnp.bfloat16),
            pltpu.VMEM((DIM, BI), jnp.bfloat16),
        ],
    )
    return pl.pallas_call(
        _grouped_body,
        grid_spec=grid_spec,
        out_shape=[
            jax.ShapeDtypeStruct((M_MAX + B, DIM), jnp.float32),
            jax.ShapeDtypeStruct((M_MAX + B, DIM), jnp.float32),
        ],
    )(scal, xs, W1, W3, W2)


def _shared_body(xb_ref, w1_ref, w3_ref, w2_ref, z_ref, w1b, w3b, w2b):
    t = pl.program_id(0)

    @pl.when(t == 0)
    def _():
        w1b[...] = w1_ref[...].astype(jnp.bfloat16)
        w3b[...] = w3_ref[...].astype(jnp.bfloat16)
        w2b[...] = w2_ref[...].astype(jnp.bfloat16)

    xt = xb_ref[pl.ds(t * BT, BT), :]
    h1 = jax.lax.dot_general(xt, w1b[...], (((1,), (1,)), ((), ())),
                             preferred_element_type=jnp.float32)
    h3 = jax.lax.dot_general(xt, w3b[...], (((1,), (1,)), ((), ())),
                             preferred_element_type=jnp.float32)
    g = (jax.nn.silu(h1) * h3).astype(jnp.bfloat16)
    o = jax.lax.dot_general(g, w2b[...], (((1,), (1,)), ((), ())),
                            preferred_element_type=jnp.float32)
    z_ref[...] = o.astype(jnp.bfloat16)


def _shared(xb, Ws1, Ws3, Ws2, jj):
    return pl.pallas_call(
        _shared_body,
        grid=(N_TOK // BT,),
        in_specs=[
            pl.BlockSpec((N_TOK, DIM), lambda t: (0, 0)),
            pl.BlockSpec((BI, DIM), lambda t, jj=jj: (jj, 0)),
            pl.BlockSpec((BI, DIM), lambda t, jj=jj: (jj, 0)),
            pl.BlockSpec((DIM, BI), lambda t, jj=jj: (0, jj)),
        ],
        out_specs=pl.BlockSpec((BT, DIM), lambda t: (t, 0)),
        out_shape=jax.ShapeDtypeStruct((N_TOK, DIM), jnp.bfloat16),
        scratch_shapes=[
            pltpu.VMEM((BI, DIM), jnp.bfloat16),
            pltpu.VMEM((BI, DIM), jnp.bfloat16),
            pltpu.VMEM((DIM, BI), jnp.bfloat16),
        ],
    )(xb, Ws1, Ws3, Ws2)


def _combine_body(wd_ref, z0_ref, z1_ref, g0a_ref, g0b_ref, g1a_ref,
                  g1b_ref, out_ref):
    wd = wd_ref[...]
    w1 = jnp.max(wd, axis=1, keepdims=True)
    w2 = jnp.sum(wd, axis=1, keepdims=True) - w1
    z = z0_ref[...].astype(jnp.float32) + z1_ref[...].astype(jnp.float32)
    y1 = g0a_ref[...] + g1a_ref[...]
    y2 = g0b_ref[...] + g1b_ref[...]
    out_ref[...] = z + w1 * y1 + w2 * y2


def _combine(wd, z0, z1, g0, g1):
    nt = N_TOK // BT
    return pl.pallas_call(
        _combine_body,
        grid=(nt,),
        in_specs=[
            pl.BlockSpec((BT, E), lambda t: (t, 0)),
            pl.BlockSpec((BT, DIM), lambda t: (t, 0)),
            pl.BlockSpec((BT, DIM), lambda t: (t, 0)),
            pl.BlockSpec((BT, DIM), lambda t: (t, 0)),
            pl.BlockSpec((BT, DIM), lambda t, nt=nt: (t + nt, 0)),
            pl.BlockSpec((BT, DIM), lambda t: (t, 0)),
            pl.BlockSpec((BT, DIM), lambda t, nt=nt: (t + nt, 0)),
        ],
        out_specs=pl.BlockSpec((BT, DIM), lambda t: (t, 0)),
        out_shape=jax.ShapeDtypeStruct((N_TOK, DIM), jnp.float32),
    )(wd, z0, z1, g0, g0, g1, g1)


def kernel(x, gate_w, W1, W2, W3, Ws1, Ws2, Ws3):
    xb = x.astype(jnp.bfloat16)
    logits = x @ gate_w.T
    wd, desta, bexp = _meta(logits)
    scal = bexp.reshape(32)
    toka = jnp.concatenate(
        [jnp.arange(N_TOK, dtype=jnp.int32)] * 2).reshape(1, NA)
    xs = _sc_dispatch(x, toka, desta)
    z0 = _shared(xb, Ws1, Ws3, Ws2, 0)
    ys0, ys1 = _grouped(scal, xs, W1, W3, W2)
    z1 = _shared(xb, Ws1, Ws3, Ws2, 1)
    g0 = _sc_gather(ys0, desta)
    g1 = _sc_gather(ys1, desta)
    return _combine(wd, z0, z1, g0, g1)


# speedup vs baseline: 3.0533x; 1.0274x over previous
import functools

import jax
import jax.numpy as jnp
from jax.experimental import pallas as pl
from jax.experimental.pallas import tpu as pltpu
from jax.experimental.pallas import tpu_sc as plsc

N_TOK = 2048
DIM = 2048
INTER = 1024
E = 8

NA = 2 * N_TOK
B = 256
M_MAX = 6144
NB = M_MAX // B
BT = 256
BI = 512
J = INTER // BI
SWIN = 16


def _fl(x):
    return x.astype(jnp.float32)


def _meta_body(l_ref, wd_ref, dest_ref, bexp_ref):
    logits = l_ref[...]
    m = jnp.max(logits, axis=1, keepdims=True)
    p = jnp.exp(logits - m)
    p = p / jnp.sum(p, axis=1, keepdims=True)
    iot = jax.lax.broadcasted_iota(jnp.int32, p.shape, 1)
    m1 = jnp.max(p, axis=1, keepdims=True)
    i1 = jnp.min(jnp.where(p == m1, iot, E), axis=1, keepdims=True)
    p2 = jnp.where(iot == i1, -jnp.inf, p)
    m2 = jnp.max(p2, axis=1, keepdims=True)
    i2 = jnp.min(jnp.where(p2 == m2, iot, E), axis=1, keepdims=True)
    wdense = jnp.where(iot == i1, m1, 0.0) + jnp.where(iot == i2, m2, 0.0)
    wd_ref[...] = wdense

    wT = jnp.transpose(wdense)
    si = jax.lax.broadcasted_iota(jnp.int32, (E, N_TOK), 0)
    t1 = jnp.max(wT, axis=0, keepdims=True)
    j1 = jnp.min(jnp.where(wT == t1, si, E), axis=0, keepdims=True)
    wr = jnp.where(si == j1, -1.0, wT)
    t2 = jnp.max(wr, axis=0, keepdims=True)
    j2 = jnp.min(jnp.where(wr == t2, si, E), axis=0, keepdims=True)
    oh1 = _fl(si == j1)
    oh2 = _fl(si == j2)
    A = jnp.concatenate([oh1, oh2], axis=1)

    A3 = A.reshape(E, NA // 128, 128)
    r128 = jax.lax.broadcasted_iota(jnp.int32, (128, 128), 0)
    c128 = jax.lax.broadcasted_iota(jnp.int32, (128, 128), 1)
    tri128 = _fl(r128 < c128)
    within = jax.lax.dot_general(A3, tri128, (((2,), (0,)), ((), ())),
                                 preferred_element_type=jnp.float32)
    cs = jnp.sum(A3, axis=2)
    nch = NA // 128
    rch = jax.lax.broadcasted_iota(jnp.int32, (nch, nch), 0)
    cch = jax.lax.broadcasted_iota(jnp.int32, (nch, nch), 1)
    trich = _fl(rch < cch)
    cpref = jax.lax.dot_general(cs, trich, (((1,), (0,)), ((), ())),
                                preferred_element_type=jnp.float32)
    rank = (within + cpref[:, :, None]).reshape(E, NA)

    counts = jnp.sum(A, axis=1, keepdims=True)
    pc = jnp.floor((counts + (B - 1)) / B) * B
    re8 = jax.lax.broadcasted_iota(jnp.int32, (E, E), 0)
    ce8 = jax.lax.broadcasted_iota(jnp.int32, (E, E), 1)
    lt8 = _fl(ce8 < re8)
    offs = jax.lax.dot_general(lt8, pc, (((1,), (0,)), ((), ())),
                               preferred_element_type=jnp.float32)
    dest = rank + offs
    desta = jnp.sum(A * dest, axis=0, keepdims=True)
    dest_ref[...] = desta.astype(jnp.int32)

    li = jax.lax.broadcasted_iota(jnp.int32, (1, 32), 1)
    bstart = _fl(li) * B
    nbelow = jnp.sum(_fl(offs <= bstart), axis=0, keepdims=True)
    bexp = nbelow - 1.0
    nact = jnp.sum(pc) / B
    row = jnp.where(li == NB, nact, bexp)
    bexp_ref[...] = row.astype(jnp.int32)


def _meta(logits):
    return pl.pallas_call(
        _meta_body,
        grid=(1,),
        in_specs=[pl.BlockSpec((N_TOK, E), lambda i: (0, 0))],
        out_specs=[
            pl.BlockSpec((N_TOK, E), lambda i: (0, 0)),
            pl.BlockSpec((1, NA), lambda i: (0, 0)),
            pl.BlockSpec((1, 32), lambda i: (0, 0)),
        ],
        out_shape=[
            jax.ShapeDtypeStruct((N_TOK, E), jnp.float32),
            jax.ShapeDtypeStruct((1, NA), jnp.int32),
            jax.ShapeDtypeStruct((1, 32), jnp.int32),
        ],
    )(logits)


IWIN = 128
NCH = IWIN // SWIN


def _vmesh():
    return plsc.VectorSubcoreMesh(
        core_axis_name="core", subcore_axis_name="subcore")


def _sc_move(data, src_idx, dst_idx, out_rows):
    @functools.partial(
        pl.kernel,
        out_type=jax.ShapeDtypeStruct((out_rows, DIM), jnp.float32),
        mesh=_vmesh(),
        scratch_types=[pltpu.VMEM((SWIN, DIM), jnp.float32)],
    )
    def k(x_hbm, src_hbm, dst_hbm, o_hbm, buf):
        def body(src_vmem, dst_vmem):
            @pl.loop(0, NCH)
            def _(c):
                sl = pl.ds(c * SWIN, SWIN)
                pltpu.sync_copy(x_hbm.at[src_vmem.at[0, sl]], buf)
                pltpu.sync_copy(buf, o_hbm.at[dst_vmem.at[0, sl]])

        pltpu.emit_pipeline(
            body,
            grid=(NA // IWIN,),
            in_specs=[
                pl.BlockSpec((1, IWIN), lambda i: (0, i)),
                pl.BlockSpec((1, IWIN), lambda i: (0, i)),
            ],
            out_specs=[],
            core_axis_name=("core", "subcore"),
            dimension_semantics=(pltpu.PARALLEL,),
        )(src_hbm, dst_hbm)

    return k(data, src_idx, dst_idx)


def _sc_dispatch(x, toka, desta):
    return _sc_move(x, toka, desta, M_MAX)


def _sc_gather(ys, desta):
    iota = jnp.arange(NA, dtype=jnp.int32).reshape(1, NA)
    return _sc_move(ys, desta, iota, NA)


def _grouped_body(s_ref, xs_ref, w1_ref, w3_ref, w2_ref, yin_ref, ys_ref,
                  w1b, w3b, w2b):
    jj = pl.program_id(0)
    b = pl.program_id(1)
    eb = s_ref[b]
    prev = s_ref[jnp.maximum(b - 1, 0)]
    changed = jnp.logical_or(b == 0, eb != prev)
    active = b < s_ref[NB]

    @pl.when(jnp.logical_and(changed, active))
    def _():
        w1b[...] = w1_ref[0].astype(jnp.bfloat16)
        w3b[...] = w3_ref[0].astype(jnp.bfloat16)
        w2b[...] = w2_ref[0].astype(jnp.bfloat16)

    @pl.when(active)
    def _():
        x = xs_ref[...].astype(jnp.bfloat16)
        h1 = jax.lax.dot_general(x, w1b[...], (((1,), (1,)), ((), ())),
                                 preferred_element_type=jnp.float32)
        h3 = jax.lax.dot_general(x, w3b[...], (((1,), (1,)), ((), ())),
                                 preferred_element_type=jnp.float32)
        g = (jax.nn.silu(h1) * h3).astype(jnp.bfloat16)
        o = jax.lax.dot_general(g, w2b[...], (((1,), (1,)), ((), ())),
                                preferred_element_type=jnp.float32)

        @pl.when(jj == 0)
        def _():
            ys_ref[...] = o

        @pl.when(jj == 1)
        def _():
            ys_ref[...] = yin_ref[...] + o


def _grouped(scal, xs, W1, W3, W2, ydummy):
    grid_spec = pltpu.PrefetchScalarGridSpec(
        num_scalar_prefetch=1,
        grid=(J, NB),
        in_specs=[
            pl.BlockSpec((B, DIM), lambda j, b, s: (b, 0)),
            pl.BlockSpec((1, BI, DIM), lambda j, b, s: (s[b], j, 0)),
            pl.BlockSpec((1, BI, DIM), lambda j, b, s: (s[b], j, 0)),
            pl.BlockSpec((1, DIM, BI), lambda j, b, s: (s[b], 0, j)),
            pl.BlockSpec((B, DIM), lambda j, b, s: (b, 0)),
        ],
        out_specs=pl.BlockSpec((B, DIM), lambda j, b, s: (b, 0)),
        scratch_shapes=[
            pltpu.VMEM((BI, DIM), jnp.bfloat16),
            pltpu.VMEM((BI, DIM), jnp.bfloat16),
            pltpu.VMEM((DIM, BI), jnp.bfloat16),
        ],
    )
    return pl.pallas_call(
        _grouped_body,
        grid_spec=grid_spec,
        out_shape=jax.ShapeDtypeStruct((M_MAX, DIM), jnp.float32),
        input_output_aliases={5: 0},
    )(scal, xs, W1, W3, W2, ydummy)


def _shared_body(carry, xb_ref, w1_ref, w3_ref, w2_ref, z_ref, w1b, w3b, w2b,
                 z0_ref=None):
    t = pl.program_id(0)

    @pl.when(t == 0)
    def _():
        w1b[...] = w1_ref[...].astype(jnp.bfloat16)
        w3b[...] = w3_ref[...].astype(jnp.bfloat16)
        w2b[...] = w2_ref[...].astype(jnp.bfloat16)

    xt = xb_ref[pl.ds(t * BT, BT), :]
    h1 = jax.lax.dot_general(xt, w1b[...], (((1,), (1,)), ((), ())),
                             preferred_element_type=jnp.float32)
    h3 = jax.lax.dot_general(xt, w3b[...], (((1,), (1,)), ((), ())),
                             preferred_element_type=jnp.float32)
    g = (jax.nn.silu(h1) * h3).astype(jnp.bfloat16)
    o = jax.lax.dot_general(g, w2b[...], (((1,), (1,)), ((), ())),
                            preferred_element_type=jnp.float32)
    if carry:
        o = o + z0_ref[...].astype(jnp.float32)
    z_ref[...] = o.astype(jnp.bfloat16)


def _shared0_body(xb_ref, w1_ref, w3_ref, w2_ref, z_ref, w1b, w3b, w2b):
    _shared_body(False, xb_ref, w1_ref, w3_ref, w2_ref, z_ref,
                 w1b, w3b, w2b)


def _shared1_body(xb_ref, w1_ref, w3_ref, w2_ref, z0_ref, z_ref,
                  w1b, w3b, w2b):
    _shared_body(True, xb_ref, w1_ref, w3_ref, w2_ref, z_ref,
                 w1b, w3b, w2b, z0_ref=z0_ref)


def _shared(xb, Ws1, Ws3, Ws2, jj, z0=None):
    in_specs = [
        pl.BlockSpec((N_TOK, DIM), lambda t: (0, 0)),
        pl.BlockSpec((BI, DIM), lambda t, jj=jj: (jj, 0)),
        pl.BlockSpec((BI, DIM), lambda t, jj=jj: (jj, 0)),
        pl.BlockSpec((DIM, BI), lambda t, jj=jj: (0, jj)),
    ]
    args = [xb, Ws1, Ws3, Ws2]
    body = _shared0_body
    if z0 is not None:
        in_specs.append(pl.BlockSpec((BT, DIM), lambda t: (t, 0)))
        args.append(z0)
        body = _shared1_body
    return pl.pallas_call(
        body,
        grid=(N_TOK // BT,),
        in_specs=in_specs,
        out_specs=pl.BlockSpec((BT, DIM), lambda t: (t, 0)),
        out_shape=jax.ShapeDtypeStruct((N_TOK, DIM), jnp.bfloat16),
        scratch_shapes=[
            pltpu.VMEM((BI, DIM), jnp.bfloat16),
            pltpu.VMEM((BI, DIM), jnp.bfloat16),
            pltpu.VMEM((DIM, BI), jnp.bfloat16),
        ],
    )(*args)


def _combine_body(wd_ref, z_ref, ga_ref, gb_ref, out_ref):
    wd = wd_ref[...]
    w1 = jnp.max(wd, axis=1, keepdims=True)
    w2 = jnp.sum(wd, axis=1, keepdims=True) - w1
    out_ref[...] = (z_ref[...].astype(jnp.float32)
                    + w1 * ga_ref[...] + w2 * gb_ref[...])


def _combine(wd, z, g):
    nt = N_TOK // BT
    return pl.pallas_call(
        _combine_body,
        grid=(nt,),
        in_specs=[
            pl.BlockSpec((BT, E), lambda t: (t, 0)),
            pl.BlockSpec((BT, DIM), lambda t: (t, 0)),
            pl.BlockSpec((BT, DIM), lambda t: (t, 0)),
            pl.BlockSpec((BT, DIM), lambda t, nt=nt: (t + nt, 0)),
        ],
        out_specs=pl.BlockSpec((BT, DIM), lambda t: (t, 0)),
        out_shape=jax.ShapeDtypeStruct((N_TOK, DIM), jnp.float32),
    )(wd, z, g, g)


def kernel(x, gate_w, W1, W2, W3, Ws1, Ws2, Ws3):
    xb = x.astype(jnp.bfloat16)
    logits = x @ gate_w.T
    wd, desta, bexp = _meta(logits)
    scal = bexp.reshape(32)
    toka = jnp.concatenate(
        [jnp.arange(N_TOK, dtype=jnp.int32)] * 2).reshape(1, NA)
    xs = _sc_dispatch(x, toka, desta)
    z0 = _shared(xb, Ws1, Ws3, Ws2, 0)
    xs, z0 = jax.lax.optimization_barrier((xs, z0))
    ydummy = jnp.zeros((M_MAX, DIM), jnp.float32)
    ys = _grouped(scal, xs, W1, W3, W2, ydummy)
    z = _shared(xb, Ws1, Ws3, Ws2, 1, z0=z0)
    g = _sc_gather(ys, desta)
    return _combine(wd, z, g)


# speedup vs baseline: 3.2988x; 1.0804x over previous
import functools

import jax
import jax.numpy as jnp
from jax.experimental import pallas as pl
from jax.experimental.pallas import tpu as pltpu
from jax.experimental.pallas import tpu_sc as plsc

N_TOK = 2048
DIM = 2048
INTER = 1024
E = 8

NA = 2 * N_TOK
B = 256
M_MAX = 6144
NB = M_MAX // B
BT = 256
BI = 512
J = INTER // BI
SWIN = 16


def _fl(x):
    return x.astype(jnp.float32)


def _meta_body(l_ref, wd_ref, dest_ref, bexp_ref):
    logits = l_ref[...]
    m = jnp.max(logits, axis=1, keepdims=True)
    p = jnp.exp(logits - m)
    p = p / jnp.sum(p, axis=1, keepdims=True)
    iot = jax.lax.broadcasted_iota(jnp.int32, p.shape, 1)
    m1 = jnp.max(p, axis=1, keepdims=True)
    i1 = jnp.min(jnp.where(p == m1, iot, E), axis=1, keepdims=True)
    p2 = jnp.where(iot == i1, -jnp.inf, p)
    m2 = jnp.max(p2, axis=1, keepdims=True)
    i2 = jnp.min(jnp.where(p2 == m2, iot, E), axis=1, keepdims=True)
    wdense = jnp.where(iot == i1, m1, 0.0) + jnp.where(iot == i2, m2, 0.0)
    wd_ref[...] = wdense

    wT = jnp.transpose(wdense)
    si = jax.lax.broadcasted_iota(jnp.int32, (E, N_TOK), 0)
    t1 = jnp.max(wT, axis=0, keepdims=True)
    j1 = jnp.min(jnp.where(wT == t1, si, E), axis=0, keepdims=True)
    wr = jnp.where(si == j1, -1.0, wT)
    t2 = jnp.max(wr, axis=0, keepdims=True)
    j2 = jnp.min(jnp.where(wr == t2, si, E), axis=0, keepdims=True)
    oh1 = _fl(si == j1)
    oh2 = _fl(si == j2)
    A = jnp.concatenate([oh1, oh2], axis=1)

    A3 = A.reshape(E, NA // 128, 128)
    r128 = jax.lax.broadcasted_iota(jnp.int32, (128, 128), 0)
    c128 = jax.lax.broadcasted_iota(jnp.int32, (128, 128), 1)
    tri128 = _fl(r128 < c128)
    within = jax.lax.dot_general(A3, tri128, (((2,), (0,)), ((), ())),
                                 preferred_element_type=jnp.float32)
    cs = jnp.sum(A3, axis=2)
    nch = NA // 128
    rch = jax.lax.broadcasted_iota(jnp.int32, (nch, nch), 0)
    cch = jax.lax.broadcasted_iota(jnp.int32, (nch, nch), 1)
    trich = _fl(rch < cch)
    cpref = jax.lax.dot_general(cs, trich, (((1,), (0,)), ((), ())),
                                preferred_element_type=jnp.float32)
    rank = (within + cpref[:, :, None]).reshape(E, NA)

    counts = jnp.sum(A, axis=1, keepdims=True)
    pc = jnp.floor((counts + (B - 1)) / B) * B
    re8 = jax.lax.broadcasted_iota(jnp.int32, (E, E), 0)
    ce8 = jax.lax.broadcasted_iota(jnp.int32, (E, E), 1)
    lt8 = _fl(ce8 < re8)
    offs = jax.lax.dot_general(lt8, pc, (((1,), (0,)), ((), ())),
                               preferred_element_type=jnp.float32)
    dest = rank + offs
    desta = jnp.sum(A * dest, axis=0, keepdims=True)
    dest_ref[...] = desta.astype(jnp.int32)

    li = jax.lax.broadcasted_iota(jnp.int32, (1, 32), 1)
    bstart = _fl(li) * B
    nbelow = jnp.sum(_fl(offs <= bstart), axis=0, keepdims=True)
    bexp = nbelow - 1.0
    nact = jnp.sum(pc) / B
    row = jnp.where(li == NB, nact, bexp)
    bexp_ref[...] = row.astype(jnp.int32)


def _meta(logits):
    return pl.pallas_call(
        _meta_body,
        grid=(1,),
        in_specs=[pl.BlockSpec((N_TOK, E), lambda i: (0, 0))],
        out_specs=[
            pl.BlockSpec((N_TOK, E), lambda i: (0, 0)),
            pl.BlockSpec((1, NA), lambda i: (0, 0)),
            pl.BlockSpec((1, 32), lambda i: (0, 0)),
        ],
        out_shape=[
            jax.ShapeDtypeStruct((N_TOK, E), jnp.float32),
            jax.ShapeDtypeStruct((1, NA), jnp.int32),
            jax.ShapeDtypeStruct((1, 32), jnp.int32),
        ],
    )(logits)


IWIN = 128
NCH = IWIN // SWIN


def _vmesh():
    return plsc.VectorSubcoreMesh(
        core_axis_name="core", subcore_axis_name="subcore")


def _sc_move(data, src_idx, dst_idx, out_rows):
    @functools.partial(
        pl.kernel,
        out_type=jax.ShapeDtypeStruct((out_rows, DIM), jnp.float32),
        mesh=_vmesh(),
        scratch_types=[pltpu.VMEM((SWIN, DIM), jnp.float32)],
    )
    def k(x_hbm, src_hbm, dst_hbm, o_hbm, buf):
        def body(src_vmem, dst_vmem):
            @pl.loop(0, NCH)
            def _(c):
                sl = pl.ds(c * SWIN, SWIN)
                pltpu.sync_copy(x_hbm.at[src_vmem.at[0, sl]], buf)
                pltpu.sync_copy(buf, o_hbm.at[dst_vmem.at[0, sl]])

        pltpu.emit_pipeline(
            body,
            grid=(NA // IWIN,),
            in_specs=[
                pl.BlockSpec((1, IWIN), lambda i: (0, i)),
                pl.BlockSpec((1, IWIN), lambda i: (0, i)),
            ],
            out_specs=[],
            core_axis_name=("core", "subcore"),
            dimension_semantics=(pltpu.PARALLEL,),
        )(src_hbm, dst_hbm)

    return k(data, src_idx, dst_idx)


def _sc_dispatch(x, toka, desta):
    return _sc_move(x, toka, desta, M_MAX)


def _sc_gather(ys, desta):
    iota = jnp.arange(NA, dtype=jnp.int32).reshape(1, NA)
    return _sc_move(ys, desta, iota, NA)


def _grouped0_body(s_ref, xs_ref, w1_ref, w3_ref, w2_ref, ys_ref,
                   w1b, w3b, w2b):
    _grouped_step(s_ref, xs_ref, w1_ref, w3_ref, w2_ref, None, ys_ref,
                  w1b, w3b, w2b)


def _grouped1_body(s_ref, xs_ref, w1_ref, w3_ref, w2_ref, yin_ref, ys_ref,
                   w1b, w3b, w2b):
    _grouped_step(s_ref, xs_ref, w1_ref, w3_ref, w2_ref, yin_ref, ys_ref,
                  w1b, w3b, w2b)


def _grouped_step(s_ref, xs_ref, w1_ref, w3_ref, w2_ref, yin_ref, ys_ref,
                  w1b, w3b, w2b):
    b = pl.program_id(0)
    eb = s_ref[b]
    prev = s_ref[jnp.maximum(b - 1, 0)]
    changed = jnp.logical_or(b == 0, eb != prev)
    active = b < s_ref[NB]

    @pl.when(jnp.logical_and(changed, active))
    def _():
        w1b[...] = w1_ref[0].astype(jnp.bfloat16)
        w3b[...] = w3_ref[0].astype(jnp.bfloat16)
        w2b[...] = w2_ref[0].astype(jnp.bfloat16)

    @pl.when(active)
    def _():
        x = xs_ref[...].astype(jnp.bfloat16)
        h1 = jax.lax.dot_general(x, w1b[...], (((1,), (1,)), ((), ())),
                                 preferred_element_type=jnp.float32)
        h3 = jax.lax.dot_general(x, w3b[...], (((1,), (1,)), ((), ())),
                                 preferred_element_type=jnp.float32)
        g = (jax.nn.silu(h1) * h3).astype(jnp.bfloat16)
        o = jax.lax.dot_general(g, w2b[...], (((1,), (1,)), ((), ())),
                                preferred_element_type=jnp.float32)
        if yin_ref is not None:
            o = yin_ref[...] + o
        ys_ref[...] = o


def _grouped_pass(scal, xs, W1, W3, W2, jj, yin=None):
    in_specs = [
        pl.BlockSpec((B, DIM), lambda b, s: (b, 0)),
        pl.BlockSpec((1, BI, DIM), lambda b, s, jj=jj: (s[b], jj, 0)),
        pl.BlockSpec((1, BI, DIM), lambda b, s, jj=jj: (s[b], jj, 0)),
        pl.BlockSpec((1, DIM, BI), lambda b, s, jj=jj: (s[b], 0, jj)),
    ]
    args = [scal, xs, W1, W3, W2]
    body = _grouped0_body
    aliases = {}
    if yin is not None:
        in_specs.append(pl.BlockSpec((B, DIM), lambda b, s: (b, 0)))
        args.append(yin)
        body = _grouped1_body
        aliases = {5: 0}
    grid_spec = pltpu.PrefetchScalarGridSpec(
        num_scalar_prefetch=1,
        grid=(NB,),
        in_specs=in_specs,
        out_specs=pl.BlockSpec((B, DIM), lambda b, s: (b, 0)),
        scratch_shapes=[
            pltpu.VMEM((BI, DIM), jnp.bfloat16),
            pltpu.VMEM((BI, DIM), jnp.bfloat16),
            pltpu.VMEM((DIM, BI), jnp.bfloat16),
        ],
    )
    return pl.pallas_call(
        body,
        grid_spec=grid_spec,
        out_shape=jax.ShapeDtypeStruct((M_MAX, DIM), jnp.float32),
        input_output_aliases=aliases,
    )(*args)


def _shared_body(carry, xb_ref, w1_ref, w3_ref, w2_ref, z_ref, w1b, w3b, w2b,
                 z0_ref=None):
    t = pl.program_id(0)

    @pl.when(t == 0)
    def _():
        w1b[...] = w1_ref[...].astype(jnp.bfloat16)
        w3b[...] = w3_ref[...].astype(jnp.bfloat16)
        w2b[...] = w2_ref[...].astype(jnp.bfloat16)

    xt = xb_ref[pl.ds(t * BT, BT), :]
    h1 = jax.lax.dot_general(xt, w1b[...], (((1,), (1,)), ((), ())),
                             preferred_element_type=jnp.float32)
    h3 = jax.lax.dot_general(xt, w3b[...], (((1,), (1,)), ((), ())),
                             preferred_element_type=jnp.float32)
    g = (jax.nn.silu(h1) * h3).astype(jnp.bfloat16)
    o = jax.lax.dot_general(g, w2b[...], (((1,), (1,)), ((), ())),
                            preferred_element_type=jnp.float32)
    if carry:
        o = o + z0_ref[...].astype(jnp.float32)
    z_ref[...] = o.astype(jnp.bfloat16)


def _shared0_body(xb_ref, w1_ref, w3_ref, w2_ref, z_ref, w1b, w3b, w2b):
    _shared_body(False, xb_ref, w1_ref, w3_ref, w2_ref, z_ref,
                 w1b, w3b, w2b)


def _shared1_body(xb_ref, w1_ref, w3_ref, w2_ref, z0_ref, z_ref,
                  w1b, w3b, w2b):
    _shared_body(True, xb_ref, w1_ref, w3_ref, w2_ref, z_ref,
                 w1b, w3b, w2b, z0_ref=z0_ref)


def _shared(xb, Ws1, Ws3, Ws2, jj, z0=None):
    in_specs = [
        pl.BlockSpec((N_TOK, DIM), lambda t: (0, 0)),
        pl.BlockSpec((BI, DIM), lambda t, jj=jj: (jj, 0)),
        pl.BlockSpec((BI, DIM), lambda t, jj=jj: (jj, 0)),
        pl.BlockSpec((DIM, BI), lambda t, jj=jj: (0, jj)),
    ]
    args = [xb, Ws1, Ws3, Ws2]
    body = _shared0_body
    if z0 is not None:
        in_specs.append(pl.BlockSpec((BT, DIM), lambda t: (t, 0)))
        args.append(z0)
        body = _shared1_body
    return pl.pallas_call(
        body,
        grid=(N_TOK // BT,),
        in_specs=in_specs,
        out_specs=pl.BlockSpec((BT, DIM), lambda t: (t, 0)),
        out_shape=jax.ShapeDtypeStruct((N_TOK, DIM), jnp.bfloat16),
        scratch_shapes=[
            pltpu.VMEM((BI, DIM), jnp.bfloat16),
            pltpu.VMEM((BI, DIM), jnp.bfloat16),
            pltpu.VMEM((DIM, BI), jnp.bfloat16),
        ],
    )(*args)


def _combine_body(wd_ref, z_ref, ga_ref, gb_ref, out_ref):
    wd = wd_ref[...]
    w1 = jnp.max(wd, axis=1, keepdims=True)
    w2 = jnp.sum(wd, axis=1, keepdims=True) - w1
    out_ref[...] = (z_ref[...].astype(jnp.float32)
                    + w1 * ga_ref[...] + w2 * gb_ref[...])


def _combine(wd, z, g):
    nt = N_TOK // BT
    return pl.pallas_call(
        _combine_body,
        grid=(nt,),
        in_specs=[
            pl.BlockSpec((BT, E), lambda t: (t, 0)),
            pl.BlockSpec((BT, DIM), lambda t: (t, 0)),
            pl.BlockSpec((BT, DIM), lambda t: (t, 0)),
            pl.BlockSpec((BT, DIM), lambda t, nt=nt: (t + nt, 0)),
        ],
        out_specs=pl.BlockSpec((BT, DIM), lambda t: (t, 0)),
        out_shape=jax.ShapeDtypeStruct((N_TOK, DIM), jnp.float32),
    )(wd, z, g, g)


def kernel(x, gate_w, W1, W2, W3, Ws1, Ws2, Ws3):
    xb = x.astype(jnp.bfloat16)
    logits = x @ gate_w.T
    wd, desta, bexp = _meta(logits)
    scal = bexp.reshape(32)
    toka = jnp.concatenate(
        [jnp.arange(N_TOK, dtype=jnp.int32)] * 2).reshape(1, NA)
    xs = _sc_dispatch(x, toka, desta)
    z0 = _shared(xb, Ws1, Ws3, Ws2, 0)
    xs, z0 = jax.lax.optimization_barrier((xs, z0))
    ys0 = _grouped_pass(scal, xs, W1, W3, W2, 0)
    ys = _grouped_pass(scal, xs, W1, W3, W2, 1, yin=ys0)
    z = _shared(xb, Ws1, Ws3, Ws2, 1, z0=z0)
    g = _sc_gather(ys, desta)
    return _combine(wd, z, g)
